# baseline (speedup 1.0000x reference)
import jax
import jax.numpy as jnp
from jax.experimental import pallas as pl

_T, _D, _E, _H = 2048, 768, 8, 768


def _gating_body(x_ref, wg_ref, gates_ref, imp_ref, load_ref):
    x = x_ref[...]
    wg = wg_ref[...]
    logits = jnp.dot(x, wg, preferred_element_type=jnp.float32)
    eidx = jax.lax.broadcasted_iota(jnp.int32, logits.shape, 1)
    m1 = jnp.max(logits, axis=1, keepdims=True)
    e1 = jnp.min(jnp.where(logits == m1, eidx, _E), axis=1, keepdims=True)
    oh1 = eidx == e1
    masked = jnp.where(oh1, -jnp.inf, logits)
    m2 = jnp.max(masked, axis=1, keepdims=True)
    e2 = jnp.min(jnp.where(masked == m2, eidx, _E), axis=1, keepdims=True)
    oh2 = eidx == e2
    b = jnp.exp(m2 - m1)
    g1 = 1.0 / (1.0 + b)
    g2 = b / (1.0 + b)
    gates = jnp.where(oh1, g1, 0.0) + jnp.where(oh2, g2, 0.0)
    gates_ref[...] = gates
    imp_ref[...] = jnp.sum(gates, axis=0, keepdims=True)
    load_ref[...] = jnp.sum((gates > 0).astype(jnp.float32), axis=0,
                            keepdims=True)


def _moe_body(x_ref, gates_ref, w1_ref, b1_ref, w2_ref, b2_ref, out_ref):
    e = pl.program_id(0)

    @pl.when(e == 0)
    def _init():
        out_ref[...] = jnp.zeros_like(out_ref)

    x = x_ref[...]
    h = jnp.dot(x, w1_ref[0], preferred_element_type=jnp.float32)
    h = jnp.maximum(h + b1_ref[0], 0.0).astype(jnp.bfloat16)
    y = jnp.dot(h, w2_ref[0], preferred_element_type=jnp.float32)
    y = y + b2_ref[0]
    gates = gates_ref[...]
    eidx = jax.lax.broadcasted_iota(jnp.int32, gates.shape, 1)
    g = jnp.sum(jnp.where(eidx == e, gates, 0.0), axis=1, keepdims=True)
    out_ref[...] += g * y


def _cv_sq(v):
    return jnp.var(v) / (jnp.mean(v) ** 2 + 1e-10)


def kernel(x, w_gate, W1, b1, W2, b2):
    gates, imp, load = pl.pallas_call(
        _gating_body,
        grid=(1,),
        in_specs=[
            pl.BlockSpec((_T, _D), lambda i: (0, 0)),
            pl.BlockSpec((_D, _E), lambda i: (0, 0)),
        ],
        out_specs=[
            pl.BlockSpec((_T, _E), lambda i: (0, 0)),
            pl.BlockSpec((1, _E), lambda i: (0, 0)),
            pl.BlockSpec((1, _E), lambda i: (0, 0)),
        ],
        out_shape=[
            jax.ShapeDtypeStruct((_T, _E), jnp.float32),
            jax.ShapeDtypeStruct((1, _E), jnp.float32),
            jax.ShapeDtypeStruct((1, _E), jnp.float32),
        ],
    )(x, w_gate)

    out = pl.pallas_call(
        _moe_body,
        grid=(_E,),
        in_specs=[
            pl.BlockSpec((_T, _D), lambda e: (0, 0)),
            pl.BlockSpec((_T, _E), lambda e: (0, 0)),
            pl.BlockSpec((1, _D, _H), lambda e: (e, 0, 0)),
            pl.BlockSpec((1, 1, _H), lambda e: (e, 0, 0)),
            pl.BlockSpec((1, _H, _D), lambda e: (e, 0, 0)),
            pl.BlockSpec((1, 1, _D), lambda e: (e, 0, 0)),
        ],
        out_specs=pl.BlockSpec((_T, _D), lambda e: (0, 0)),
        out_shape=jax.ShapeDtypeStruct((_T, _D), jnp.float32),
    )(x.astype(jnp.bfloat16), gates, W1.astype(jnp.bfloat16),
      b1.reshape(_E, 1, _H), W2.astype(jnp.bfloat16),
      b2.reshape(_E, 1, _D))

    moe_loss = _cv_sq(imp[0]) + _cv_sq(load[0])
    return out, moe_loss


# speedup vs baseline: 1.2522x; 1.2522x over previous
import jax
import jax.numpy as jnp
from jax.experimental import pallas as pl

_T, _D, _E, _H = 2048, 768, 8, 768


def _gating_body(x_ref, wg_ref, gates_ref, imp_ref, load_ref):
    x = x_ref[...]
    wg = wg_ref[...]
    logits = jnp.dot(x, wg, preferred_element_type=jnp.float32)
    eidx = jax.lax.broadcasted_iota(jnp.int32, logits.shape, 1)
    m1 = jnp.max(logits, axis=1, keepdims=True)
    e1 = jnp.min(jnp.where(logits == m1, eidx, _E), axis=1, keepdims=True)
    oh1 = eidx == e1
    masked = jnp.where(oh1, -jnp.inf, logits)
    m2 = jnp.max(masked, axis=1, keepdims=True)
    e2 = jnp.min(jnp.where(masked == m2, eidx, _E), axis=1, keepdims=True)
    oh2 = eidx == e2
    b = jnp.exp(m2 - m1)
    g1 = 1.0 / (1.0 + b)
    g2 = b / (1.0 + b)
    gates = jnp.where(oh1, g1, 0.0) + jnp.where(oh2, g2, 0.0)
    gates_ref[...] = gates
    imp_ref[...] = jnp.sum(gates, axis=0, keepdims=True)
    load_ref[...] = jnp.sum((gates > 0).astype(jnp.float32), axis=0,
                            keepdims=True)


def _moe_body(x_ref, gates_ref, w1_ref, b1_ref, w2_ref, b2_ref, out_ref):
    e = pl.program_id(0)

    @pl.when(e == 0)
    def _init():
        out_ref[...] = jnp.zeros_like(out_ref)

    x = x_ref[...].astype(jnp.bfloat16)
    h = jnp.dot(x, w1_ref[0].astype(jnp.bfloat16),
                preferred_element_type=jnp.float32)
    h = jnp.maximum(h + b1_ref[0], 0.0).astype(jnp.bfloat16)
    y = jnp.dot(h, w2_ref[0].astype(jnp.bfloat16),
                preferred_element_type=jnp.float32)
    y = y + b2_ref[0]
    gates = gates_ref[...]
    eidx = jax.lax.broadcasted_iota(jnp.int32, gates.shape, 1)
    g = jnp.sum(jnp.where(eidx == e, gates, 0.0), axis=1, keepdims=True)
    out_ref[...] += g * y


def _cv_sq(v):
    return jnp.var(v) / (jnp.mean(v) ** 2 + 1e-10)


def kernel(x, w_gate, W1, b1, W2, b2):
    gates, imp, load = pl.pallas_call(
        _gating_body,
        grid=(1,),
        in_specs=[
            pl.BlockSpec((_T, _D), lambda i: (0, 0)),
            pl.BlockSpec((_D, _E), lambda i: (0, 0)),
        ],
        out_specs=[
            pl.BlockSpec((_T, _E), lambda i: (0, 0)),
            pl.BlockSpec((1, _E), lambda i: (0, 0)),
            pl.BlockSpec((1, _E), lambda i: (0, 0)),
        ],
        out_shape=[
            jax.ShapeDtypeStruct((_T, _E), jnp.float32),
            jax.ShapeDtypeStruct((1, _E), jnp.float32),
            jax.ShapeDtypeStruct((1, _E), jnp.float32),
        ],
    )(x, w_gate)

    out = pl.pallas_call(
        _moe_body,
        grid=(_E,),
        in_specs=[
            pl.BlockSpec((_T, _D), lambda e: (0, 0)),
            pl.BlockSpec((_T, _E), lambda e: (0, 0)),
            pl.BlockSpec((1, _D, _H), lambda e: (e, 0, 0)),
            pl.BlockSpec((1, 1, _H), lambda e: (e, 0, 0)),
            pl.BlockSpec((1, _H, _D), lambda e: (e, 0, 0)),
            pl.BlockSpec((1, 1, _D), lambda e: (e, 0, 0)),
        ],
        out_specs=pl.BlockSpec((_T, _D), lambda e: (0, 0)),
        out_shape=jax.ShapeDtypeStruct((_T, _D), jnp.float32),
    )(x, gates, W1, b1.reshape(_E, 1, _H), W2, b2.reshape(_E, 1, _D))

    moe_loss = _cv_sq(imp[0]) + _cv_sq(load[0])
    return out, moe_loss


# speedup vs baseline: 1.2535x; 1.0010x over previous
import jax
import jax.numpy as jnp
from jax.experimental import pallas as pl

_T, _D, _E, _H = 2048, 768, 8, 768


def _gating_body(x_ref, wg_ref, gates_ref, imp_ref, load_ref):
    x = x_ref[...]
    wg = wg_ref[...]
    logits = jnp.dot(x, wg, preferred_element_type=jnp.float32)
    eidx = jax.lax.broadcasted_iota(jnp.int32, logits.shape, 1)
    m1 = jnp.max(logits, axis=1, keepdims=True)
    e1 = jnp.min(jnp.where(logits == m1, eidx, _E), axis=1, keepdims=True)
    oh1 = eidx == e1
    masked = jnp.where(oh1, -jnp.inf, logits)
    m2 = jnp.max(masked, axis=1, keepdims=True)
    e2 = jnp.min(jnp.where(masked == m2, eidx, _E), axis=1, keepdims=True)
    oh2 = eidx == e2
    b = jnp.exp(m2 - m1)
    g1 = 1.0 / (1.0 + b)
    g2 = b / (1.0 + b)
    gates = jnp.where(oh1, g1, 0.0) + jnp.where(oh2, g2, 0.0)
    gates_ref[...] = gates
    imp_ref[...] = jnp.sum(gates, axis=0, keepdims=True)
    load_ref[...] = jnp.sum((gates > 0).astype(jnp.float32), axis=0,
                            keepdims=True)


def _moe_body(x_ref, gates_ref, w1_ref, b1_ref, w2_ref, b2_ref, out_ref):
    e = pl.program_id(0)

    @pl.when(e == 0)
    def _init():
        out_ref[...] = jnp.zeros_like(out_ref)

    x = x_ref[...]
    h = jnp.dot(x, w1_ref[0], preferred_element_type=jnp.float32)
    h = jnp.maximum(h + b1_ref[0], 0.0)
    y = jnp.dot(h, w2_ref[0], preferred_element_type=jnp.float32)
    y = y + b2_ref[0]
    gates = gates_ref[...]
    eidx = jax.lax.broadcasted_iota(jnp.int32, gates.shape, 1)
    g = jnp.sum(jnp.where(eidx == e, gates, 0.0), axis=1, keepdims=True)
    out_ref[...] += g * y


def _cv_sq(v):
    return jnp.var(v) / (jnp.mean(v) ** 2 + 1e-10)


def kernel(x, w_gate, W1, b1, W2, b2):
    gates, imp, load = pl.pallas_call(
        _gating_body,
        grid=(1,),
        in_specs=[
            pl.BlockSpec((_T, _D), lambda i: (0, 0)),
            pl.BlockSpec((_D, _E), lambda i: (0, 0)),
        ],
        out_specs=[
            pl.BlockSpec((_T, _E), lambda i: (0, 0)),
            pl.BlockSpec((1, _E), lambda i: (0, 0)),
            pl.BlockSpec((1, _E), lambda i: (0, 0)),
        ],
        out_shape=[
            jax.ShapeDtypeStruct((_T, _E), jnp.float32),
            jax.ShapeDtypeStruct((1, _E), jnp.float32),
            jax.ShapeDtypeStruct((1, _E), jnp.float32),
        ],
    )(x, w_gate)

    out = pl.pallas_call(
        _moe_body,
        grid=(_E,),
        in_specs=[
            pl.BlockSpec((_T, _D), lambda e: (0, 0)),
            pl.BlockSpec((_T, _E), lambda e: (0, 0)),
            pl.BlockSpec((1, _D, _H), lambda e: (e, 0, 0)),
            pl.BlockSpec((1, 1, _H), lambda e: (e, 0, 0)),
            pl.BlockSpec((1, _H, _D), lambda e: (e, 0, 0)),
            pl.BlockSpec((1, 1, _D), lambda e: (e, 0, 0)),
        ],
        out_specs=pl.BlockSpec((_T, _D), lambda e: (0, 0)),
        out_shape=jax.ShapeDtypeStruct((_T, _D), jnp.float32),
    )(x, gates, W1, b1.reshape(_E, 1, _H), W2, b2.reshape(_E, 1, _D))

    moe_loss = _cv_sq(imp[0]) + _cv_sq(load[0])
    return out, moe_loss
